# baseline (speedup 1.0000x reference)
import functools

import jax
import jax.numpy as jnp
from jax.experimental import pallas as pl
from jax.experimental.pallas import tpu as pltpu

T = 2048
D = 1024
E = 8
F = 512
BT = 512
NI = T // BT


def _moe_body(x_ref, gw_ref, w1_ref, w3_ref, w2_ref, out_ref, comb_ref):
    e = pl.program_id(0)
    i = pl.program_id(1)

    @pl.when(jnp.logical_and(e == 0, i == 0))
    def _router():
        logits = jax.lax.dot_general(
            x_ref[...], gw_ref[...], (((1,), (1,)), ((), ())),
            preferred_element_type=jnp.float32)
        m = jnp.max(logits, axis=1, keepdims=True)
        ex = jnp.exp(logits - m)
        probs = ex / jnp.sum(ex, axis=1, keepdims=True)
        lane = jax.lax.broadcasted_iota(jnp.int32, (T, E), 1)
        rank = jnp.zeros((T, E), jnp.int32)
        for j in range(E):
            pj = probs[:, j:j + 1]
            beats = (pj > probs) | ((pj == probs) & (j < lane))
            rank = rank + beats.astype(jnp.int32)
        comb_ref[...] = probs * (rank < 2).astype(jnp.float32)

    xs = x_ref[pl.ds(i * BT, BT), :]
    h1 = jax.lax.dot_general(xs, w1_ref[0], (((1,), (0,)), ((), ())),
                             preferred_element_type=jnp.float32)
    h3 = jax.lax.dot_general(xs, w3_ref[0], (((1,), (0,)), ((), ())),
                             preferred_element_type=jnp.float32)
    h = h1 * jax.nn.sigmoid(h1) * h3
    y = jax.lax.dot_general(h, w2_ref[0], (((1,), (0,)), ((), ())),
                            preferred_element_type=jnp.float32)

    cb = comb_ref[pl.ds(i * BT, BT), :]
    lane = jax.lax.broadcasted_iota(jnp.int32, (BT, E), 1)
    w_e = jnp.sum(jnp.where(lane == e, cb, 0.0), axis=1, keepdims=True)
    yw = y * w_e

    @pl.when(e == 0)
    def _init():
        out_ref[pl.ds(i * BT, BT), :] = yw

    @pl.when(e != 0)
    def _acc():
        out_ref[pl.ds(i * BT, BT), :] = out_ref[pl.ds(i * BT, BT), :] + yw


def _moe(hidden_states, gate_w, w1, w3, w2):
    x = hidden_states.astype(jnp.float32)
    w1t = jnp.swapaxes(w1, 1, 2)
    w3t = jnp.swapaxes(w3, 1, 2)
    w2t = jnp.swapaxes(w2, 1, 2)
    out = pl.pallas_call(
        _moe_body,
        grid=(E, NI),
        in_specs=[
            pl.BlockSpec((T, D), lambda e, i: (0, 0)),
            pl.BlockSpec((E, D), lambda e, i: (0, 0)),
            pl.BlockSpec((1, D, F), lambda e, i: (e, 0, 0)),
            pl.BlockSpec((1, D, F), lambda e, i: (e, 0, 0)),
            pl.BlockSpec((1, F, D), lambda e, i: (e, 0, 0)),
        ],
        out_specs=pl.BlockSpec((T, D), lambda e, i: (0, 0)),
        out_shape=jax.ShapeDtypeStruct((T, D), jnp.float32),
        scratch_shapes=[pltpu.VMEM((T, E), jnp.float32)],
        compiler_params=pltpu.CompilerParams(
            dimension_semantics=("arbitrary", "arbitrary")),
    )(x, gate_w.astype(jnp.float32), w1t, w3t, w2t)
    return out


def kernel(hidden_states, num_global_tokens, max_num_tokens_per_gpu,
           gate_w, w1, w3, w2):
    del num_global_tokens, max_num_tokens_per_gpu
    return _moe(hidden_states, gate_w, w1, w3, w2)


# speedup vs baseline: 1.2213x; 1.2213x over previous
import functools

import jax
import jax.numpy as jnp
from jax.experimental import pallas as pl
from jax.experimental.pallas import tpu as pltpu

T = 2048
D = 1024
E = 8
F = 512
BT = 512
NI = T // BT


def _moe_body(x_ref, gw_ref, w1_ref, w3_ref, w2_ref, out_ref, comb_ref,
              xbf_ref):
    e = pl.program_id(0)
    i = pl.program_id(1)

    @pl.when(jnp.logical_and(e == 0, i == 0))
    def _router():
        logits = jax.lax.dot_general(
            x_ref[...], gw_ref[...], (((1,), (1,)), ((), ())),
            preferred_element_type=jnp.float32)
        m = jnp.max(logits, axis=1, keepdims=True)
        ex = jnp.exp(logits - m)
        probs = ex / jnp.sum(ex, axis=1, keepdims=True)
        lane = jax.lax.broadcasted_iota(jnp.int32, (T, E), 1)
        rank = jnp.zeros((T, E), jnp.int32)
        for j in range(E):
            pj = probs[:, j:j + 1]
            beats = (pj > probs) | ((pj == probs) & (j < lane))
            rank = rank + beats.astype(jnp.int32)
        comb_ref[...] = probs * (rank < 2).astype(jnp.float32)

    @pl.when(e == 0)
    def _cast():
        xbf_ref[pl.ds(i * BT, BT), :] = (
            x_ref[pl.ds(i * BT, BT), :].astype(jnp.bfloat16))

    xs = xbf_ref[pl.ds(i * BT, BT), :]
    h1 = jax.lax.dot_general(xs, w1_ref[0], (((1,), (0,)), ((), ())),
                             preferred_element_type=jnp.float32)
    h3 = jax.lax.dot_general(xs, w3_ref[0], (((1,), (0,)), ((), ())),
                             preferred_element_type=jnp.float32)
    h = (h1 * jax.nn.sigmoid(h1) * h3).astype(jnp.bfloat16)
    y = jax.lax.dot_general(h, w2_ref[0], (((1,), (0,)), ((), ())),
                            preferred_element_type=jnp.float32)

    cb = comb_ref[pl.ds(i * BT, BT), :]
    lane = jax.lax.broadcasted_iota(jnp.int32, (BT, E), 1)
    w_e = jnp.sum(jnp.where(lane == e, cb, 0.0), axis=1, keepdims=True)
    yw = y * w_e

    @pl.when(e == 0)
    def _init():
        out_ref[pl.ds(i * BT, BT), :] = yw

    @pl.when(e != 0)
    def _acc():
        out_ref[pl.ds(i * BT, BT), :] = out_ref[pl.ds(i * BT, BT), :] + yw


def _moe(hidden_states, gate_w, w1, w3, w2):
    x = hidden_states.astype(jnp.float32)
    w1t = jnp.swapaxes(w1, 1, 2).astype(jnp.bfloat16)
    w3t = jnp.swapaxes(w3, 1, 2).astype(jnp.bfloat16)
    w2t = jnp.swapaxes(w2, 1, 2).astype(jnp.bfloat16)
    out = pl.pallas_call(
        _moe_body,
        grid=(E, NI),
        in_specs=[
            pl.BlockSpec((T, D), lambda e, i: (0, 0)),
            pl.BlockSpec((E, D), lambda e, i: (0, 0)),
            pl.BlockSpec((1, D, F), lambda e, i: (e, 0, 0)),
            pl.BlockSpec((1, D, F), lambda e, i: (e, 0, 0)),
            pl.BlockSpec((1, F, D), lambda e, i: (e, 0, 0)),
        ],
        out_specs=pl.BlockSpec((T, D), lambda e, i: (0, 0)),
        out_shape=jax.ShapeDtypeStruct((T, D), jnp.float32),
        scratch_shapes=[pltpu.VMEM((T, E), jnp.float32),
                        pltpu.VMEM((T, D), jnp.bfloat16)],
        compiler_params=pltpu.CompilerParams(
            dimension_semantics=("arbitrary", "arbitrary")),
    )(x, gate_w.astype(jnp.float32), w1t, w3t, w2t)
    return out


def kernel(hidden_states, num_global_tokens, max_num_tokens_per_gpu,
           gate_w, w1, w3, w2):
    del num_global_tokens, max_num_tokens_per_gpu
    return _moe(hidden_states, gate_w, w1, w3, w2)
